# baseline (speedup 1.0000x reference)
import functools

import jax
import jax.numpy as jnp
from jax import lax
from jax.experimental import pallas as pl
from jax.experimental.pallas import tpu as pltpu

_N_EMBED = 1024
_EMBED_DIM = 64
_BETA = 0.25
_ROWS = 16384
_TILE = 512
_GRID = _ROWS // _TILE


def _l2n(x, eps=1e-12):
    n = jnp.linalg.norm(x, ord=2, axis=-1, keepdims=True)
    return x / jnp.maximum(n, eps)


def _vq_body(zn_ref, z_ref, zsq_ref, ent_ref, esq_ref, emb_ref,
             qst_ref, idx_ref, loss_ref):
    i = pl.program_id(0)
    zn = zn_ref[...]
    s = lax.dot_general(zn, ent_ref[...], (((1,), (0,)), ((), ())),
                        precision=lax.Precision.DEFAULT,
                        preferred_element_type=jnp.float32)
    d2 = (zsq_ref[...] - 2.0 * s) + esq_ref[...]
    dist = jnp.sqrt(jnp.maximum(d2, 0.0))
    m = jnp.min(dist, axis=1, keepdims=True)
    iota = lax.broadcasted_iota(jnp.int32, (_TILE, _N_EMBED), 1)
    big = jnp.int32(_N_EMBED)
    idx = jnp.min(jnp.where(dist == m, iota, big), axis=1, keepdims=True)
    idx_ref[...] = idx

    onehot = (iota == idx).astype(jnp.float32)
    q = lax.dot_general(onehot, emb_ref[...], (((1,), (0,)), ((), ())),
                        precision=lax.Precision.HIGHEST,
                        preferred_element_type=jnp.float32)
    z = z_ref[...]
    qst_ref[...] = z + (q - z)

    part = jnp.sum(m * m)

    @pl.when(i == 0)
    def _init():
        loss_ref[0, 0] = 0.0

    loss_ref[0, 0] += part

    @pl.when(i == _GRID - 1)
    def _fin():
        loss_ref[0, 0] = loss_ref[0, 0] * jnp.float32(
            _BETA / (_ROWS * _EMBED_DIM))


@jax.jit
def kernel(z, embed_weight):
    flat_z = z.reshape(-1, _EMBED_DIM)
    zn = _l2n(flat_z)
    en = _l2n(embed_weight)
    zsq = jnp.sum(zn * zn, axis=1, keepdims=True)
    esq = jnp.sum(en * en, axis=1)[None, :]
    ent = en.T

    grid_spec = pl.GridSpec(
        grid=(_GRID,),
        in_specs=[
            pl.BlockSpec((_TILE, _EMBED_DIM), lambda i: (i, 0)),
            pl.BlockSpec((_TILE, _EMBED_DIM), lambda i: (i, 0)),
            pl.BlockSpec((_TILE, 1), lambda i: (i, 0)),
            pl.BlockSpec((_EMBED_DIM, _N_EMBED), lambda i: (0, 0)),
            pl.BlockSpec((1, _N_EMBED), lambda i: (0, 0)),
            pl.BlockSpec((_N_EMBED, _EMBED_DIM), lambda i: (0, 0)),
        ],
        out_specs=[
            pl.BlockSpec((_TILE, _EMBED_DIM), lambda i: (i, 0)),
            pl.BlockSpec((_TILE, 1), lambda i: (i, 0)),
            pl.BlockSpec(memory_space=pltpu.SMEM),
        ],
    )
    qst, idx, loss = pl.pallas_call(
        _vq_body,
        grid_spec=grid_spec,
        out_shape=[
            jax.ShapeDtypeStruct((_ROWS, _EMBED_DIM), jnp.float32),
            jax.ShapeDtypeStruct((_ROWS, 1), jnp.int32),
            jax.ShapeDtypeStruct((1, 1), jnp.float32),
        ],
    )(zn, flat_z, zsq, ent, esq, embed_weight)

    quantized_st = qst.reshape(z.shape)
    encoding_indices = idx.reshape(z.shape[:-1])
    vq_loss = loss[0, 0]
    return quantized_st, encoding_indices, vq_loss


# speedup vs baseline: 1.4576x; 1.4576x over previous
import jax
import jax.numpy as jnp
from jax import lax
from jax.experimental import pallas as pl
from jax.experimental.pallas import tpu as pltpu

_N_EMBED = 1024
_EMBED_DIM = 64
_BETA = 0.25
_ROWS = 16384
_TILE = 1024
_GRID = _ROWS // _TILE


def _l2n(x, eps=1e-12):
    n = jnp.linalg.norm(x, ord=2, axis=-1, keepdims=True)
    return x / jnp.maximum(n, eps)


def _vq_body(zn_ref, z_ref, zsq_ref, ent_ref, esq_ref, emb3_ref,
             qst_ref, idx_ref, loss_ref):
    i = pl.program_id(0)
    zn = zn_ref[...]
    s = lax.dot_general(zn, ent_ref[...], (((1,), (0,)), ((), ())),
                        precision=lax.Precision.DEFAULT,
                        preferred_element_type=jnp.float32)
    d2 = (zsq_ref[...] - 2.0 * s) + esq_ref[...]
    dist = jnp.sqrt(jnp.maximum(d2, 0.0))
    m = jnp.min(dist, axis=1, keepdims=True)
    iota = lax.broadcasted_iota(jnp.int32, (_TILE, _N_EMBED), 1)
    big = jnp.int32(_N_EMBED)
    idx = jnp.min(jnp.where(dist == m, iota, big), axis=1, keepdims=True)
    idx_ref[...] = idx

    onehot = (iota == idx).astype(jnp.bfloat16)
    q3 = lax.dot_general(onehot, emb3_ref[...], (((1,), (0,)), ((), ())),
                         precision=lax.Precision.DEFAULT,
                         preferred_element_type=jnp.float32)
    q = (q3[:, :_EMBED_DIM] + q3[:, _EMBED_DIM:2 * _EMBED_DIM]
         + q3[:, 2 * _EMBED_DIM:])
    z = z_ref[...]
    qst_ref[...] = z + (q - z)

    part = jnp.sum(m * m)

    @pl.when(i == 0)
    def _init():
        loss_ref[0, 0] = 0.0

    loss_ref[0, 0] += part

    @pl.when(i == _GRID - 1)
    def _fin():
        loss_ref[0, 0] = loss_ref[0, 0] * jnp.float32(
            _BETA / (_ROWS * _EMBED_DIM))


@jax.jit
def kernel(z, embed_weight):
    flat_z = z.reshape(-1, _EMBED_DIM)
    zn = _l2n(flat_z)
    en = _l2n(embed_weight)
    zsq = jnp.sum(zn * zn, axis=1, keepdims=True)
    esq = jnp.sum(en * en, axis=1)[None, :]
    ent = en.T

    e1 = embed_weight.astype(jnp.bfloat16)
    r1 = embed_weight - e1.astype(jnp.float32)
    e2 = r1.astype(jnp.bfloat16)
    e3 = (r1 - e2.astype(jnp.float32)).astype(jnp.bfloat16)
    emb3 = jnp.concatenate([e1, e2, e3], axis=1)

    grid_spec = pl.GridSpec(
        grid=(_GRID,),
        in_specs=[
            pl.BlockSpec((_TILE, _EMBED_DIM), lambda i: (i, 0)),
            pl.BlockSpec((_TILE, _EMBED_DIM), lambda i: (i, 0)),
            pl.BlockSpec((_TILE, 1), lambda i: (i, 0)),
            pl.BlockSpec((_EMBED_DIM, _N_EMBED), lambda i: (0, 0)),
            pl.BlockSpec((1, _N_EMBED), lambda i: (0, 0)),
            pl.BlockSpec((_N_EMBED, 3 * _EMBED_DIM), lambda i: (0, 0)),
        ],
        out_specs=[
            pl.BlockSpec((_TILE, _EMBED_DIM), lambda i: (i, 0)),
            pl.BlockSpec((_TILE, 1), lambda i: (i, 0)),
            pl.BlockSpec(memory_space=pltpu.SMEM),
        ],
    )
    qst, idx, loss = pl.pallas_call(
        _vq_body,
        grid_spec=grid_spec,
        out_shape=[
            jax.ShapeDtypeStruct((_ROWS, _EMBED_DIM), jnp.float32),
            jax.ShapeDtypeStruct((_ROWS, 1), jnp.int32),
            jax.ShapeDtypeStruct((1, 1), jnp.float32),
        ],
    )(zn, flat_z, zsq, ent, esq, emb3)

    quantized_st = qst.reshape(z.shape)
    encoding_indices = idx.reshape(z.shape[:-1])
    vq_loss = loss[0, 0]
    return quantized_st, encoding_indices, vq_loss
